# baseline (device time: 91649 ns/iter reference)
import jax
import jax.numpy as jnp
from jax import lax
from jax.experimental import pallas as pl
from jax.experimental.pallas import tpu as pltpu

N_DEV = 8
N_HOPS = N_DEV // 2
N_SEG = 4


def kernel(x):
    m_per, n = x.shape
    seg = m_per // N_SEG
    last = N_HOPS - 1

    def gray(t):
        t = t % N_DEV
        return jnp.where(t < 4, t, 11 - t)

    def fwd_segs(h):
        return range(N_SEG) if h < last else range(N_SEG // 2)

    def bwd_segs(h):
        return range(N_SEG) if h < last else range(N_SEG // 2, N_SEG)

    def body(x_ref, out_ref, fwd_send, fwd_recv, bwd_send, bwd_recv):
        my_pos = lax.axis_index("i")
        r = gray(my_pos)
        nxt = gray(r + 1)
        prv = gray(r - 1)

        barrier_sem = pltpu.get_barrier_semaphore()
        for nbr in (nxt, prv):
            pl.semaphore_signal(
                barrier_sem, inc=1,
                device_id=(nbr,), device_id_type=pl.DeviceIdType.MESH,
            )
        pl.semaphore_wait(barrier_sem, 2)

        def desc(src, row0, sems_s, sems_r, h, s, target):
            return pltpu.make_async_remote_copy(
                src_ref=src,
                dst_ref=out_ref.at[pl.ds(row0, seg), :],
                send_sem=sems_s.at[h, s],
                recv_sem=sems_r.at[h, s],
                device_id=(target,),
                device_id_type=pl.DeviceIdType.MESH,
            )

        df, db = {}, {}
        for h in range(N_HOPS):
            for s in fwd_segs(h):
                if h == 0:
                    row0 = my_pos * m_per + s * seg
                    src = x_ref.at[pl.ds(s * seg, seg), :]
                else:
                    row0 = gray(r - h) * m_per + s * seg
                    src = out_ref.at[pl.ds(row0, seg), :]
                df[h, s] = desc(src, row0, fwd_send, fwd_recv, h, s, nxt)
            for s in bwd_segs(h):
                if h == 0:
                    row0 = my_pos * m_per + s * seg
                    src = x_ref.at[pl.ds(s * seg, seg), :]
                else:
                    row0 = gray(r + h) * m_per + s * seg
                    src = out_ref.at[pl.ds(row0, seg), :]
                db[h, s] = desc(src, row0, bwd_send, bwd_recv, h, s, prv)

        for s in range(N_SEG):
            df[0, s].start()
            db[0, s].start()
        out_ref[pl.ds(my_pos * m_per, m_per), :] = x_ref[:, :]

        for h in range(1, N_HOPS):
            for sf, sb in zip(fwd_segs(h), bwd_segs(h)):
                df[h - 1, sf].wait_recv()
                df[h, sf].start()
                db[h - 1, sb].wait_recv()
                db[h, sb].start()

        for s in range(N_SEG // 2, N_SEG):
            df[last - 1, s].wait_recv()
        for s in range(N_SEG // 2):
            db[last - 1, s].wait_recv()
        for s in fwd_segs(last):
            df[last, s].wait_recv()
        for s in bwd_segs(last):
            db[last, s].wait_recv()

        for d in (*df.values(), *db.values()):
            d.wait_send()

    return pl.pallas_call(
        body,
        out_shape=jax.ShapeDtypeStruct((N_DEV * m_per, n), x.dtype),
        in_specs=[pl.BlockSpec(memory_space=pltpu.VMEM)],
        out_specs=pl.BlockSpec(memory_space=pltpu.VMEM),
        scratch_shapes=[
            pltpu.SemaphoreType.DMA((N_HOPS, N_SEG)),
            pltpu.SemaphoreType.DMA((N_HOPS, N_SEG)),
            pltpu.SemaphoreType.DMA((N_HOPS, N_SEG)),
            pltpu.SemaphoreType.DMA((N_HOPS, N_SEG)),
        ],
        compiler_params=pltpu.CompilerParams(collective_id=0),
    )(x)


# device time: 70696 ns/iter; 1.2964x vs baseline; 1.2964x over previous
import jax
import jax.numpy as jnp
from jax import lax
from jax.experimental import pallas as pl
from jax.experimental.pallas import tpu as pltpu

N_DEV = 8
N_SEG = 4


def kernel(x):
    m_per, n = x.shape
    seg = m_per // N_SEG
    ta = (m_per // 3 + 7) // 8 * 8
    tb = ta
    tc = m_per - ta - tb

    def gray(t):
        t = t % N_DEV
        return jnp.where(t < 4, t, 11 - t)

    def body(x_ref, out_ref, fs, fr, bs, br, ps, pr):
        my_pos = lax.axis_index("i")
        r = gray(my_pos)
        nxt = gray(r + 1)
        prv = gray(r - 1)
        sgn = jnp.where(r % 2 == 1, 1, -1)
        par = gray(r - 3 * sgn)

        barrier_sem = pltpu.get_barrier_semaphore()
        for nbr in (nxt, prv, par):
            pl.semaphore_signal(
                barrier_sem, inc=1,
                device_id=(nbr,), device_id_type=pl.DeviceIdType.MESH,
            )
        pl.semaphore_wait(barrier_sem, 3)

        def desc(src, row0, rows, sems_s, sems_r, i, j, target):
            return pltpu.make_async_remote_copy(
                src_ref=src,
                dst_ref=out_ref.at[pl.ds(row0, rows), :],
                send_sem=sems_s.at[i, j],
                recv_sem=sems_r.at[i, j],
                device_id=(target,),
                device_id_type=pl.DeviceIdType.MESH,
            )

        def own(sems_s, sems_r, s, target):
            return desc(
                x_ref.at[pl.ds(s * seg, seg), :],
                my_pos * m_per + s * seg, seg, sems_s, sems_r, 0, s, target,
            )

        def fwd_chunk(origin_ring, slot, s, sems_s, sems_r, target):
            row0 = gray(origin_ring) * m_per + s * seg
            return desc(
                out_ref.at[pl.ds(row0, seg), :],
                row0, seg, sems_s, sems_r, slot, s, target,
            )

        def third(origin_ring, off, rows, sems_s, sems_r, slot, target):
            row0 = gray(origin_ring) * m_per + off
            return desc(
                out_ref.at[pl.ds(row0, rows), :],
                row0, rows, sems_s, sems_r, slot, 0, target,
            )

        F0 = [own(fs, fr, s, nxt) for s in range(N_SEG)]
        B0 = [own(bs, br, s, prv) for s in range(N_SEG)]
        P0 = [own(ps, pr, s, par) for s in range(N_SEG)]
        F1 = [fwd_chunk(r - 1, 1, s, fs, fr, nxt) for s in range(N_SEG)]
        B1 = [fwd_chunk(r + 1, 1, s, bs, br, prv) for s in range(N_SEG)]
        P2 = [fwd_chunk(r + 2 * sgn, 2, s, ps, pr, par) for s in range(N_SEG)]
        F2 = third(r - 3, 0, ta, fs, fr, 2, nxt)
        B2 = third(r + 3, ta, tb, bs, br, 2, prv)
        P1 = third(r + sgn, ta + tb, tc, ps, pr, 1, par)

        for s in range(N_SEG):
            F0[s].start()
            B0[s].start()
            P0[s].start()
        out_ref[pl.ds(my_pos * m_per, m_per), :] = x_ref[:, :]

        for s in range(N_SEG):
            F0[s].wait_recv()
            F1[s].start()
            B0[s].wait_recv()
            B1[s].start()
        P1.start()

        for s in range(N_SEG):
            F1[s].wait_recv()
            B1[s].wait_recv()
            P2[s].start()

        for s in range(N_SEG):
            P0[s].wait_recv()
        P2[0].wait_recv()
        P2[1].wait_recv()
        F2.start()
        P2[2].wait_recv()
        B2.start()
        P2[3].wait_recv()

        F2.wait_recv()
        B2.wait_recv()
        P1.wait_recv()

        for d in (*F0, *B0, *P0, *F1, *B1, *P2, F2, B2, P1):
            d.wait_send()

    return pl.pallas_call(
        body,
        out_shape=jax.ShapeDtypeStruct((N_DEV * m_per, n), x.dtype),
        in_specs=[pl.BlockSpec(memory_space=pltpu.VMEM)],
        out_specs=pl.BlockSpec(memory_space=pltpu.VMEM),
        scratch_shapes=[
            pltpu.SemaphoreType.DMA((3, N_SEG)),
            pltpu.SemaphoreType.DMA((3, N_SEG)),
            pltpu.SemaphoreType.DMA((3, N_SEG)),
            pltpu.SemaphoreType.DMA((3, N_SEG)),
            pltpu.SemaphoreType.DMA((3, N_SEG)),
            pltpu.SemaphoreType.DMA((3, N_SEG)),
        ],
        compiler_params=pltpu.CompilerParams(collective_id=0),
    )(x)


# device time: 67730 ns/iter; 1.3532x vs baseline; 1.0438x over previous
import jax
import jax.numpy as jnp
from jax import lax
from jax.experimental import pallas as pl
from jax.experimental.pallas import tpu as pltpu

N_DEV = 8
N_SEG = 4


def kernel(x):
    m_per, n = x.shape
    seg = m_per // N_SEG
    ta = (m_per // 3 + 7) // 8 * 8
    tb = ta
    tc = m_per - ta - tb

    def gray(t):
        t = t % N_DEV
        return jnp.where(t < 4, t, 11 - t)

    def body(x_ref, out_ref, fs, fr, bs, br, ps, pr):
        my_pos = lax.axis_index("i")
        r = gray(my_pos)
        nxt = gray(r + 1)
        prv = gray(r - 1)
        sgn = jnp.where(r % 2 == 1, 1, -1)
        par = gray(r - 3 * sgn)

        barrier_sem = pltpu.get_barrier_semaphore()
        for nbr in (nxt, prv, par):
            pl.semaphore_signal(
                barrier_sem, inc=1,
                device_id=(nbr,), device_id_type=pl.DeviceIdType.MESH,
            )
        pl.semaphore_wait(barrier_sem, 3)

        def desc(src, row0, rows, sems_s, sems_r, i, j, target):
            return pltpu.make_async_remote_copy(
                src_ref=src,
                dst_ref=out_ref.at[pl.ds(row0, rows), :],
                send_sem=sems_s.at[i, j],
                recv_sem=sems_r.at[i, j],
                device_id=(target,),
                device_id_type=pl.DeviceIdType.MESH,
            )

        def own(sems_s, sems_r, s, target):
            return desc(
                x_ref.at[pl.ds(s * seg, seg), :],
                my_pos * m_per + s * seg, seg, sems_s, sems_r, 0, s, target,
            )

        def fwd_chunk(origin_ring, slot, s, sems_s, sems_r, target):
            row0 = gray(origin_ring) * m_per + s * seg
            return desc(
                out_ref.at[pl.ds(row0, seg), :],
                row0, seg, sems_s, sems_r, slot, s, target,
            )

        def third(origin_ring, off, rows, sems_s, sems_r, slot, target):
            row0 = gray(origin_ring) * m_per + off
            return desc(
                out_ref.at[pl.ds(row0, rows), :],
                row0, rows, sems_s, sems_r, slot, 0, target,
            )

        F0 = [own(fs, fr, s, nxt) for s in range(N_SEG)]
        B0 = [own(bs, br, s, prv) for s in range(N_SEG)]
        P0 = [own(ps, pr, s, par) for s in range(N_SEG)]
        F1 = [fwd_chunk(r - 1, 1, s, fs, fr, nxt) for s in range(N_SEG)]
        B1 = [fwd_chunk(r + 1, 1, s, bs, br, prv) for s in range(N_SEG)]
        P2 = [fwd_chunk(r + 2 * sgn, 2, s, ps, pr, par) for s in range(N_SEG)]
        off_f = jnp.where(sgn < 0, 0, ta)
        off_b = ta - off_f
        F2 = third(r - 3, off_f, ta, fs, fr, 2, nxt)
        B2 = third(r + 3, off_b, tb, bs, br, 2, prv)
        P1 = third(r + sgn, ta + tb, tc, ps, pr, 1, par)

        for s in range(N_SEG):
            F0[s].start()
            B0[s].start()
            P0[s].start()
        out_ref[pl.ds(my_pos * m_per, m_per), :] = x_ref[:, :]

        for s in range(N_SEG):
            F0[s].wait_recv()
            F1[s].start()
            B0[s].wait_recv()
            B1[s].start()
        P1.start()

        for s in range(N_SEG):
            F1[s].wait_recv()
            B1[s].wait_recv()
            P2[s].start()

        for s in range(N_SEG):
            P0[s].wait_recv()
        P2[0].wait_recv()
        P2[1].wait_recv()
        F2.start()
        B2.start()
        P2[2].wait_recv()
        P2[3].wait_recv()

        F2.wait_recv()
        B2.wait_recv()
        P1.wait_recv()

        for d in (*F0, *B0, *P0, *F1, *B1, *P2, F2, B2, P1):
            d.wait_send()

    return pl.pallas_call(
        body,
        out_shape=jax.ShapeDtypeStruct((N_DEV * m_per, n), x.dtype),
        in_specs=[pl.BlockSpec(memory_space=pltpu.VMEM)],
        out_specs=pl.BlockSpec(memory_space=pltpu.VMEM),
        scratch_shapes=[
            pltpu.SemaphoreType.DMA((3, N_SEG)),
            pltpu.SemaphoreType.DMA((3, N_SEG)),
            pltpu.SemaphoreType.DMA((3, N_SEG)),
            pltpu.SemaphoreType.DMA((3, N_SEG)),
            pltpu.SemaphoreType.DMA((3, N_SEG)),
            pltpu.SemaphoreType.DMA((3, N_SEG)),
        ],
        compiler_params=pltpu.CompilerParams(collective_id=0),
    )(x)


# device time: 67693 ns/iter; 1.3539x vs baseline; 1.0005x over previous
import jax
import jax.numpy as jnp
from jax import lax
from jax.experimental import pallas as pl
from jax.experimental.pallas import tpu as pltpu

N_DEV = 8
N_SEG = 4


def kernel(x):
    m_per, n = x.shape
    seg = m_per // N_SEG
    ta = (m_per // 3 + 7) // 8 * 8
    tb = ta
    tc = m_per - ta - tb

    def gray(t):
        t = t % N_DEV
        return jnp.where(t < 4, t, 11 - t)

    def body(x_ref, out_ref, fs, fr, bs, br, ps, pr, cp_sem):
        my_pos = lax.axis_index("i")
        r = gray(my_pos)
        nxt = gray(r + 1)
        prv = gray(r - 1)
        sgn = jnp.where(r % 2 == 1, 1, -1)
        par = gray(r - 3 * sgn)

        barrier_sem = pltpu.get_barrier_semaphore()
        for nbr in (nxt, prv, par):
            pl.semaphore_signal(
                barrier_sem, inc=1,
                device_id=(nbr,), device_id_type=pl.DeviceIdType.MESH,
            )
        pl.semaphore_wait(barrier_sem, 3)

        def desc(src, row0, rows, sems_s, sems_r, i, j, target):
            return pltpu.make_async_remote_copy(
                src_ref=src,
                dst_ref=out_ref.at[pl.ds(row0, rows), :],
                send_sem=sems_s.at[i, j],
                recv_sem=sems_r.at[i, j],
                device_id=(target,),
                device_id_type=pl.DeviceIdType.MESH,
            )

        def own(sems_s, sems_r, s, target):
            return desc(
                x_ref.at[pl.ds(s * seg, seg), :],
                my_pos * m_per + s * seg, seg, sems_s, sems_r, 0, s, target,
            )

        def fwd_chunk(origin_ring, slot, s, sems_s, sems_r, target):
            row0 = gray(origin_ring) * m_per + s * seg
            return desc(
                out_ref.at[pl.ds(row0, seg), :],
                row0, seg, sems_s, sems_r, slot, s, target,
            )

        def third(origin_ring, off, rows, sems_s, sems_r, slot, target):
            row0 = gray(origin_ring) * m_per + off
            return desc(
                out_ref.at[pl.ds(row0, rows), :],
                row0, rows, sems_s, sems_r, slot, 0, target,
            )

        F0 = [own(fs, fr, s, nxt) for s in range(N_SEG)]
        B0 = [own(bs, br, s, prv) for s in range(N_SEG)]
        P0 = [own(ps, pr, s, par) for s in range(N_SEG)]
        F1 = [fwd_chunk(r - 1, 1, s, fs, fr, nxt) for s in range(N_SEG)]
        B1 = [fwd_chunk(r + 1, 1, s, bs, br, prv) for s in range(N_SEG)]
        P2 = [fwd_chunk(r + 2 * sgn, 2, s, ps, pr, par) for s in range(N_SEG)]
        off_f = jnp.where(sgn < 0, 0, ta)
        off_b = ta - off_f
        F2 = third(r - 3, off_f, ta, fs, fr, 2, nxt)
        B2 = third(r + 3, off_b, tb, bs, br, 2, prv)
        P1 = third(r + sgn, ta + tb, tc, ps, pr, 1, par)

        for s in range(N_SEG):
            F0[s].start()
            B0[s].start()
            P0[s].start()
        local = pltpu.make_async_copy(
            x_ref, out_ref.at[pl.ds(my_pos * m_per, m_per), :], cp_sem
        )
        local.start()

        for s in range(N_SEG):
            F0[s].wait_recv()
            F1[s].start()
            B0[s].wait_recv()
            B1[s].start()
        P1.start()

        for s in range(N_SEG):
            F1[s].wait_recv()
            B1[s].wait_recv()
            P2[s].start()

        for s in range(N_SEG):
            P0[s].wait_recv()
        P2[0].wait_recv()
        P2[1].wait_recv()
        F2.start()
        B2.start()
        P2[2].wait_recv()
        P2[3].wait_recv()

        F2.wait_recv()
        B2.wait_recv()
        P1.wait_recv()

        local.wait()

        for d in (*F0, *B0, *P0, *F1, *B1, *P2, F2, B2, P1):
            d.wait_send()

    return pl.pallas_call(
        body,
        out_shape=jax.ShapeDtypeStruct((N_DEV * m_per, n), x.dtype),
        in_specs=[pl.BlockSpec(memory_space=pl.ANY)],
        out_specs=pl.BlockSpec(memory_space=pl.ANY),
        scratch_shapes=[
            pltpu.SemaphoreType.DMA((3, N_SEG)),
            pltpu.SemaphoreType.DMA((3, N_SEG)),
            pltpu.SemaphoreType.DMA((3, N_SEG)),
            pltpu.SemaphoreType.DMA((3, N_SEG)),
            pltpu.SemaphoreType.DMA((3, N_SEG)),
            pltpu.SemaphoreType.DMA((3, N_SEG)),
            pltpu.SemaphoreType.DMA,
        ],
        compiler_params=pltpu.CompilerParams(collective_id=0),
    )(x)
